# baseline (speedup 1.0000x reference)
import functools

import jax
import jax.numpy as jnp
from jax.experimental import pallas as pl
from jax.experimental.pallas import tpu as pltpu

_E = 8
_TOP_K = 2
_T = 32
_H = 2048
_I = 4096
_TH = 256
_NH = _H // _TH
_TI = 512
_NI = _I // _TI
_NT = 2 * _I // _TI


def _routing_weights(idx, aff):
    t, e = aff.shape
    erange = jax.lax.broadcasted_iota(jnp.int32, (t, e), 1)
    chosen = jnp.zeros((t, e), dtype=jnp.bool_)
    for k in range(idx.shape[1]):
        chosen = chosen | (idx[:, k][:, None] == erange)
    aff_m = jnp.where(chosen, aff, 0.0)
    denom = jnp.maximum(jnp.sum(jnp.abs(aff_m), axis=1, keepdims=True), 1e-12)
    return aff_m / denom


def _mlp_kernel(idx_ref, aff_ref, x_ref, gu_ref, down_ref, out_ref, acc_ref):
    e = pl.program_id(0)
    p = pl.program_id(1)

    @pl.when(p < _NH)
    def _gate_up_phase():
        part = jnp.dot(x_ref[:, :], gu_ref[0],
                       preferred_element_type=jnp.float32)
        for j in range(_NT):
            tile = part[:, j * _TI:(j + 1) * _TI]
            @pl.when(p == 0)
            def _():
                acc_ref[j] = tile
            @pl.when(p > 0)
            def _():
                acc_ref[j] += tile

    @pl.when(p >= _NH)
    def _down_phase():
        w = _routing_weights(idx_ref[:, :], aff_ref[:, :])
        ecol = jax.lax.broadcasted_iota(jnp.int32, w.shape, 1)
        we = jnp.sum(jnp.where(ecol == e, w, 0.0), axis=1, keepdims=True)

        i = p - _NH
        g = acc_ref[i]
        u = acc_ref[_NI + i]
        inter = (g * jax.nn.sigmoid(g)) * u * we
        contrib = jnp.dot(inter, down_ref[0],
                          preferred_element_type=jnp.float32)

        @pl.when((e == 0) & (p == _NH))
        def _init():
            out_ref[:, :] = jnp.zeros_like(out_ref)

        out_ref[:, :] += contrib


@functools.partial(jax.jit, static_argnames=())
def kernel(hidden_states, expert_affinities, expert_index, gate_up_proj, down_proj):
    idx = expert_index.astype(jnp.int32)
    grid = (_E, _NH + _NI)
    return pl.pallas_call(
        _mlp_kernel,
        grid=grid,
        in_specs=[
            pl.BlockSpec((_T, _TOP_K), lambda e, p: (0, 0)),
            pl.BlockSpec((_T, _E), lambda e, p: (0, 0)),
            pl.BlockSpec((_T, _TH), lambda e, p: (0, jnp.minimum(p, _NH - 1))),
            pl.BlockSpec((1, _TH, 2 * _I),
                         lambda e, p: (e, jnp.minimum(p, _NH - 1), 0)),
            pl.BlockSpec((1, _TI, _H),
                         lambda e, p: (e, jnp.maximum(p - _NH, 0), 0)),
        ],
        out_specs=pl.BlockSpec((_T, _H), lambda e, p: (0, 0)),
        out_shape=jax.ShapeDtypeStruct((_T, _H), jnp.float32),
        scratch_shapes=[pltpu.VMEM((_NT, _T, _TI), jnp.float32)],
    )(idx, expert_affinities, hidden_states, gate_up_proj, down_proj)


# speedup vs baseline: 1.0563x; 1.0563x over previous
import functools

import jax
import jax.numpy as jnp
from jax.experimental import pallas as pl
from jax.experimental.pallas import tpu as pltpu
from jax.experimental.pallas import tpu_sc as plsc

_E = 8
_TOP_K = 2
_T = 32
_H = 2048
_I = 4096
_TS = 512
_NI = _I // _TS


_SC_CORES = 2
_SC_SUBCORES = 16
_SC_LANES = 16


def _make_routing_sc():
    nc, nl = _SC_CORES, _SC_LANES

    mesh = plsc.VectorSubcoreMesh(
        core_axis_name="c", subcore_axis_name="s",
        num_cores=_SC_CORES, num_subcores=_SC_SUBCORES)

    @functools.partial(
        pl.kernel, mesh=mesh,
        out_type=jax.ShapeDtypeStruct((_T * _E,), jnp.float32),
        compiler_params=pltpu.CompilerParams(needs_layout_passes=False),
        scratch_types=[
            pltpu.VMEM((nl,), jnp.int32),
            pltpu.VMEM((nl,), jnp.float32),
            pltpu.VMEM((nl,), jnp.float32),
        ],
    )
    def routing_sc(idx_hbm, aff_hbm, out_hbm, idx_v, aff_v, w_v):
        tok = jax.lax.axis_index("s") * nc + jax.lax.axis_index("c")

        @pl.when(tok < _T)
        def _():
            chunk = tok // (nl // _TOP_K)
            p0 = _TOP_K * (tok % (nl // _TOP_K))
            pltpu.sync_copy(idx_hbm.at[pl.ds(chunk * nl, nl)], idx_v)
            pltpu.sync_copy(aff_hbm.at[pl.ds(tok * _E, nl)], aff_v)

            v = idx_v[...]
            eid = jax.lax.iota(jnp.int32, nl)
            i0 = jnp.sum(jnp.where(eid == p0, v, 0))
            i1 = jnp.sum(jnp.where(eid == p0 + 1, v, 0))
            chosen = ((i0 == eid) | (i1 == eid)) & (eid < _E)
            a = aff_v[...]
            m = jnp.where(chosen, a, 0.0)
            denom = jnp.maximum(jnp.sum(jnp.abs(m)), 1e-12)
            w_v[...] = m / denom
            pltpu.sync_copy(w_v.at[pl.ds(0, _E)], out_hbm.at[pl.ds(tok * _E, _E)])

    return routing_sc


_routing_sc = _make_routing_sc()


def _mlp_kernel(w_ref, x_ref, gate_ref, up_ref, down_ref, out_ref):
    e = pl.program_id(0)
    i = pl.program_id(1)

    w = w_ref[:, :]
    ecol = jax.lax.broadcasted_iota(jnp.int32, w.shape, 1)
    we = jnp.sum(jnp.where(ecol == e, w, 0.0), axis=1, keepdims=True)

    x = x_ref[:, :]
    g = jnp.dot(x, gate_ref[0], preferred_element_type=jnp.float32)
    u = jnp.dot(x, up_ref[0], preferred_element_type=jnp.float32)
    inter = (g * jax.nn.sigmoid(g)) * u * we
    contrib = jnp.dot(inter, down_ref[0], preferred_element_type=jnp.float32)

    @pl.when((e == 0) & (i == 0))
    def _init():
        out_ref[:, :] = jnp.zeros_like(out_ref)

    out_ref[:, :] += contrib


@functools.partial(jax.jit, static_argnames=())
def kernel(hidden_states, expert_affinities, expert_index, gate_up_proj, down_proj):
    idx_flat = expert_index.astype(jnp.int32).reshape(-1)
    aff_pad = jnp.concatenate(
        [expert_affinities.reshape(-1),
         jnp.zeros((_SC_LANES,), jnp.float32)])
    w_flat = _routing_sc(idx_flat, aff_pad)
    w = w_flat.reshape(_T, _E)

    grid = (_E, _NI)
    return pl.pallas_call(
        _mlp_kernel,
        grid=grid,
        in_specs=[
            pl.BlockSpec((_T, _E), lambda e, i: (0, 0)),
            pl.BlockSpec((_T, _H), lambda e, i: (0, 0)),
            pl.BlockSpec((1, _H, _TS), lambda e, i: (e, 0, i)),
            pl.BlockSpec((1, _H, _TS), lambda e, i: (e, 0, _NI + i)),
            pl.BlockSpec((1, _TS, _H), lambda e, i: (e, i, 0)),
        ],
        out_specs=pl.BlockSpec((_T, _H), lambda e, i: (0, 0)),
        out_shape=jax.ShapeDtypeStruct((_T, _H), jnp.float32),
    )(w, hidden_states, gate_up_proj, gate_up_proj, down_proj)
